# baseline (speedup 1.0000x reference)
import functools

import jax
import jax.numpy as jnp
from jax import lax
from jax.experimental import pallas as pl
from jax.experimental.pallas import tpu as pltpu
from jax.experimental.pallas import tpu_sc as plsc

_B, _T, _D = 4, 8192, 1024
_BH = _B // 2
_NC, _NS = 2, 16
_NW = _NC * _NS
_RPW = _T // _NW
_CH = 64
_NCHUNK = _RPW // _CH
_BT = 512


def _make_sc_broadcast():
  mesh = plsc.VectorSubcoreMesh(core_axis_name="c", subcore_axis_name="s")

  @functools.partial(
      pl.kernel,
      out_type=jax.ShapeDtypeStruct((_BH, _T, _D), jnp.float32),
      mesh=mesh,
      scratch_types=[
          pltpu.VMEM((_CH, _D), jnp.float32),
          pltpu.SemaphoreType.DMA,
      ],
  )
  def body(params_hbm, out_hbm, buf, sem):
    wid = lax.axis_index("s") * _NC + lax.axis_index("c")
    for k in range(_NCHUNK):
      base = wid * _RPW + k * _CH
      pltpu.sync_copy(params_hbm.at[pl.ds(base, _CH)], buf)
      copies = [
          pltpu.async_copy(buf, out_hbm.at[n, pl.ds(base, _CH)], sem)
          for n in range(_BH)
      ]
      for cp in copies:
        cp.wait()

  return body


_sc_broadcast = _make_sc_broadcast()


def _tc_body(p_ref, o_ref):
  o_ref[0] = p_ref[...]


_tc_broadcast = pl.pallas_call(
    _tc_body,
    grid=(_T // _BT, _BH),
    in_specs=[pl.BlockSpec((_BT, _D), lambda t, n: (t, 0))],
    out_specs=pl.BlockSpec((1, _BT, _D), lambda t, n: (n, t, 0)),
    out_shape=jax.ShapeDtypeStruct((_BH, _T, _D), jnp.float32),
)


@jax.jit
def kernel(x, params):
  del x
  lo = _sc_broadcast(params)
  hi = _tc_broadcast(params)
  return jnp.concatenate([lo, hi], axis=0)


# speedup vs baseline: 2.3245x; 2.3245x over previous
import functools

import jax
import jax.numpy as jnp
from jax import lax
from jax.experimental import pallas as pl
from jax.experimental.pallas import tpu as pltpu
from jax.experimental.pallas import tpu_sc as plsc

_B, _T, _D = 4, 8192, 1024
_NC, _NS = 2, 16
_NW = _NC * _NS
_RPW = _T // _NW
_CH = 64
_NCHUNK = _RPW // _CH


def _make_sc_broadcast():
  mesh = plsc.VectorSubcoreMesh(core_axis_name="c", subcore_axis_name="s")

  @functools.partial(
      pl.kernel,
      out_type=jax.ShapeDtypeStruct((_B, _T, _D), jnp.float32),
      mesh=mesh,
      scratch_types=[
          pltpu.VMEM((_CH, _D), jnp.float32),
          pltpu.VMEM((_CH, _D), jnp.float32),
          pltpu.SemaphoreType.DMA,
          pltpu.SemaphoreType.DMA,
      ],
  )
  def body(params_hbm, out_hbm, buf0, buf1, gsem, ssem):
    bufs = (buf0, buf1)
    wid = lax.axis_index("s") * _NC + lax.axis_index("c")
    base0 = wid * _RPW
    gathers = [None, None]
    scatters = [None, None]
    gathers[0] = pltpu.async_copy(
        params_hbm.at[pl.ds(base0, _CH)], bufs[0], gsem)
    for k in range(_NCHUNK):
      b = k % 2
      gathers[b].wait()
      if k + 1 < _NCHUNK:
        nb = (k + 1) % 2
        if scatters[nb] is not None:
          for cp in scatters[nb]:
            cp.wait()
          scatters[nb] = None
        gathers[nb] = pltpu.async_copy(
            params_hbm.at[pl.ds(base0 + (k + 1) * _CH, _CH)], bufs[nb], gsem)
      scatters[b] = [
          pltpu.async_copy(
              bufs[b], out_hbm.at[n, pl.ds(base0 + k * _CH, _CH)], ssem)
          for n in range(_B)
      ]
    for b in range(2):
      if scatters[b] is not None:
        for cp in scatters[b]:
          cp.wait()

  return body


_sc_broadcast = _make_sc_broadcast()


@jax.jit
def kernel(x, params):
  del x
  return _sc_broadcast(params)


# speedup vs baseline: 2.3497x; 1.0108x over previous
import functools

import jax
import jax.numpy as jnp
from jax import lax
from jax.experimental import pallas as pl
from jax.experimental.pallas import tpu as pltpu
from jax.experimental.pallas import tpu_sc as plsc

_B, _T, _D = 4, 8192, 1024
_NC, _NS = 2, 16
_NW = _NC * _NS
_RPW = _T // _NW
_CH = 64
_NCHUNK = _RPW // _CH


def _make_sc_broadcast():
  mesh = plsc.VectorSubcoreMesh(core_axis_name="c", subcore_axis_name="s")

  @functools.partial(
      pl.kernel,
      out_type=jax.ShapeDtypeStruct((_B, _T, _D), jnp.float32),
      mesh=mesh,
      scratch_types=[
          pltpu.VMEM((_CH, _D), jnp.float32),
          pltpu.VMEM((_CH, _D), jnp.float32),
          pltpu.SemaphoreType.DMA,
          pltpu.SemaphoreType.DMA,
      ],
  )
  def body(params_hbm, out_hbm, buf0, buf1, gsem, ssem):
    bufs = (buf0, buf1)
    wid = lax.axis_index("s") * _NC + lax.axis_index("c")
    base0 = wid * _RPW
    gathers = [
        pltpu.async_copy(
            params_hbm.at[pl.ds(base0 + j * _CH, _CH)], bufs[j], gsem)
        for j in range(2)
    ]
    scatters = [None, None]
    for k in range(_NCHUNK):
      b = k % 2
      gathers[b].wait()
      scatters[b] = [
          pltpu.async_copy(
              bufs[b], out_hbm.at[n, pl.ds(base0 + k * _CH, _CH)], ssem)
          for n in range(_B)
      ]
      if k + 2 < _NCHUNK:
        for cp in scatters[b]:
          cp.wait()
        scatters[b] = None
        gathers[b] = pltpu.async_copy(
            params_hbm.at[pl.ds(base0 + (k + 2) * _CH, _CH)], bufs[b], gsem)
    for b in range(2):
      if scatters[b] is not None:
        for cp in scatters[b]:
          cp.wait()

  return body


_sc_broadcast = _make_sc_broadcast()


@jax.jit
def kernel(x, params):
  del x
  return _sc_broadcast(params)
